# Initial kernel scaffold; baseline (speedup 1.0000x reference)
import jax
import jax.numpy as jnp
from jax.experimental import pallas as pl


def kernel(x, edge_index, batch, params):
    raise NotImplementedError("write your pallas kernel here")


# speedup vs baseline: 1.6282x; 1.6282x over previous
import functools

import numpy as np
import jax
import jax.numpy as jnp
from jax import lax
from jax.experimental import pallas as pl
from jax.experimental.pallas import tpu as pltpu
from jax.experimental.pallas import tpu_sc as plsc

N = 10000
NPAD = 10240
E = 160000
NC = 2
NS = 16
NW = NC * NS
C = 128
CSC = 64
K = 125
EW = E // NW
NB = EW // K
MB = 1280
NMB = NPAD // MB


def _make_edge_kernel(n, nchunk, c, nc, ns, k, nb):
  npr = n // ns
  zr = 128
  nz = npr // zr
  assert npr % zr == 0
  mesh = plsc.VectorSubcoreMesh(
      core_axis_name="c", subcore_axis_name="s",
      num_cores=nc, num_subcores=ns)

  @functools.partial(
      pl.kernel,
      out_type=jax.ShapeDtypeStruct((nc, nchunk, n, c), jnp.float32),
      mesh=mesh,
      compiler_params=pltpu.CompilerParams(use_tc_tiling_on_sc=False),
      scratch_types=[
          pltpu.VMEM((nb, 1, k), jnp.int32),
          pltpu.VMEM((nb, 1, k), jnp.int32),
          pltpu.VMEM((k, c), jnp.float32),
          pltpu.VMEM((k, c), jnp.float32),
          pltpu.VMEM((k, c), jnp.float32),
          pltpu.VMEM((zr, c), jnp.float32),
          pltpu.VMEM_SHARED((n, c), jnp.float32),
          pltpu.SemaphoreType.DMA,
          pltpu.SemaphoreType.DMA,
      ],
  )
  def edge_kernel(uv, srcg, dstg, out, srcv, dstv, ub, vb, rb, zbuf, s_sh,
                  sem1, sem2):
    cid = lax.axis_index("c")
    sid = lax.axis_index("s")
    wid = sid * nc + cid
    pltpu.sync_copy(srcg.at[wid], srcv)
    pltpu.sync_copy(dstg.at[wid], dstv)

    def zrow(i, carry):
      for c8 in range(c // 16):
        zbuf[i, pl.ds(c8 * 16, 16)] = jnp.zeros((16,), jnp.float32)
      return carry
    lax.fori_loop(0, zr, zrow, 0)

    def chunk(ch, ccarry):
      for z in range(nz):
        pltpu.sync_copy(zbuf, s_sh.at[pl.ds(sid * npr + z * zr, zr)])
      plsc.subcore_barrier()

      def batch(j, carry):
        cp1 = pltpu.async_copy(uv.at[ch].at[dstv.at[j, 0]], ub, sem1)
        cp2 = pltpu.async_copy(uv.at[nchunk + ch].at[srcv.at[j, 0]], vb, sem2)
        cp1.wait()
        cp2.wait()

        def row(i, rcarry):
          for c8 in range(c // 16):
            s = pl.ds(c8 * 16, 16)
            rb[i, s] = jnp.maximum(ub[i, s] + vb[i, s], 0.0)
          return rcarry
        lax.fori_loop(0, k, row, 0)
        pltpu.sync_copy(rb, s_sh.at[dstv.at[j, 0]], add=True)
        return carry
      lax.fori_loop(0, nb, batch, 0)
      plsc.subcore_barrier()

      pltpu.sync_copy(s_sh.at[pl.ds(sid * npr, npr)],
                      out.at[cid, ch, pl.ds(sid * npr, npr)])
      plsc.subcore_barrier()
      return ccarry
    lax.fori_loop(0, nchunk, chunk, 0)

  return edge_kernel


def _uv_embed_body(x_ref, w_ref, b_ref, o_ref):
  xb = x_ref[...]
  p9 = xb[:, 0:9]
  feats = [xb]
  feats += [jnp.sin(p9 * (2.0 ** f)) for f in range(10)]
  feats += [jnp.cos(p9 * (2.0 ** f)) for f in range(10)]
  h = jnp.concatenate(feats, axis=1)
  r = jnp.dot(h, w_ref[...], preferred_element_type=jnp.float32, precision=lax.Precision.HIGHEST) + b_ref[0]
  o_ref[0] = r[:, :CSC]
  o_ref[1] = r[:, CSC:]


def _uv_norm_body(h_ref, sc_ref, sh_ref, w_ref, b_ref, o_ref):
  h = jnp.maximum(h_ref[...] * sc_ref[...] + sh_ref[...], 0.0)
  r = jnp.dot(h, w_ref[...], preferred_element_type=jnp.float32, precision=lax.Precision.HIGHEST) + b_ref[0]
  o_ref[0] = r[:, :CSC]
  o_ref[1] = r[:, CSC:]


def _uv_call(h, scale, shift, w, b3, embed):
  n, d = h.shape
  nchunk2 = w.shape[1] // C
  if embed:
    body = _uv_embed_body
    in_arrays = (h, w, b3)
    in_specs = [
        pl.BlockSpec((MB, d), lambda m, j: (m, 0)),
        pl.BlockSpec((w.shape[0], C), lambda m, j: (0, j)),
        pl.BlockSpec((1, 1, C), lambda m, j: (j, 0, 0)),
    ]
  else:
    body = _uv_norm_body
    in_arrays = (h, scale, shift, w, b3)
    in_specs = [
        pl.BlockSpec((MB, d), lambda m, j: (m, 0)),
        pl.BlockSpec((1, d), lambda m, j: (0, 0)),
        pl.BlockSpec((1, d), lambda m, j: (0, 0)),
        pl.BlockSpec((d, C), lambda m, j: (0, j)),
        pl.BlockSpec((1, 1, C), lambda m, j: (j, 0, 0)),
    ]
  return pl.pallas_call(
      body,
      grid=(NMB, nchunk2),
      in_specs=in_specs,
      out_specs=pl.BlockSpec((2, MB, CSC), lambda m, j: (j, m, 0)),
      out_shape=jax.ShapeDtypeStruct((2 * nchunk2, n, CSC), jnp.float32),
  )(*in_arrays)


def _make_out_body(nchunk, dout):
  def body(s_ref, cnt_ref, w_ref, b_ref, o_ref, st_ref):
    acc = jnp.zeros((MB, dout), jnp.float32)
    for ch in range(nchunk // 2):
      t = jnp.concatenate(
          [s_ref[0, 2 * ch] + s_ref[1, 2 * ch],
           s_ref[0, 2 * ch + 1] + s_ref[1, 2 * ch + 1]], axis=1)
      acc = acc + jnp.dot(t, w_ref[pl.ds(ch * C, C), :],
                          preferred_element_type=jnp.float32,
                          precision=lax.Precision.HIGHEST)
    cnt = cnt_ref[...]
    outb = (acc + cnt * b_ref[...]) / jnp.maximum(cnt, 1.0)
    o_ref[...] = outb
    s0 = jnp.sum(outb, axis=0, keepdims=True)
    s1 = jnp.sum(outb * outb, axis=0, keepdims=True)
    st = jnp.concatenate([s0, s1], axis=0)

    @pl.when(pl.program_id(0) == 0)
    def _init():
      st_ref[...] = st

    @pl.when(pl.program_id(0) != 0)
    def _acc():
      st_ref[...] = st_ref[...] + st
  return body


def _out_call(sp, cnt, w2, b2):
  nc, nchunk, n, c = sp.shape
  dh, dout = w2.shape
  return pl.pallas_call(
      _make_out_body(nchunk, dout),
      grid=(NMB,),
      in_specs=[
          pl.BlockSpec((nc, nchunk, MB, c), lambda m: (0, 0, m, 0)),
          pl.BlockSpec((MB, 1), lambda m: (m, 0)),
          pl.BlockSpec((dh, dout), lambda m: (0, 0)),
          pl.BlockSpec((1, dout), lambda m: (0, 0)),
      ],
      out_specs=[
          pl.BlockSpec((MB, dout), lambda m: (m, 0)),
          pl.BlockSpec((2, dout), lambda m: (0, 0)),
      ],
      out_shape=[
          jax.ShapeDtypeStruct((n, dout), jnp.float32),
          jax.ShapeDtypeStruct((2, dout), jnp.float32),
      ],
  )(sp, cnt, w2, b2)


def _build_perm():
  perm = np.zeros(196, np.int32)
  for m in range(196):
    if m < 9:
      b, r = divmod(m, 3)
      perm[m] = b * 63 + r
    elif m < 16:
      perm[m] = 189 + (m - 9)
    elif m < 106:
      f, kk = divmod(m - 16, 9)
      b, r = divmod(kk, 3)
      perm[m] = b * 63 + 3 + f * 3 + r
    else:
      f, kk = divmod(m - 106, 9)
      b, r = divmod(kk, 3)
      perm[m] = b * 63 + 33 + f * 3 + r
  return perm

_PERM = _build_perm()


def _prep_layer(params, name, perm=None):
  w1 = params[name + '_W1']
  d = w1.shape[0] // 2
  dh = w1.shape[1]
  wa, wb = w1[:d], w1[d:]
  wuv = jnp.concatenate([wa - wb, wb], axis=1)
  if perm is not None:
    wuv = wuv[perm, :]
  b1 = params[name + '_b1']
  bcat = jnp.concatenate([b1, jnp.zeros_like(b1)])
  b3 = bcat.reshape(2 * dh // C, 1, C)
  return wuv, b3, dh


def kernel(x, edge_index, batch, params):
  del batch
  src = edge_index[0].astype(jnp.int32)
  dst = edge_index[1].astype(jnp.int32)
  srcg = src.reshape(NW, NB, 1, K)
  dstg = dst.reshape(NW, NB, 1, K)

  uv_cnt = jnp.concatenate(
      [jnp.zeros((1, NPAD, CSC), jnp.float32),
       jnp.ones((1, NPAD, CSC), jnp.float32)],
      axis=0)
  cnt_kernel = _make_edge_kernel(NPAD, 1, CSC, NC, NS, K, NB)
  cp = cnt_kernel(uv_cnt, srcg, dstg)
  cnt = cp[0, 0, :, 0:1] + cp[1, 0, :, 0:1]

  layer_specs = [
      ('gc1', 'n1'), ('gc2', 'n2'), ('gc3', 'n3'), ('gc4', 'n4'),
      ('gc5', None),
  ]
  edge_kernels = {}
  h = jnp.pad(x, ((0, NPAD - N), (0, 0)))
  stats = None
  prev_norm = None
  for name, norm in layer_specs:
    wuv, b3, dh = _prep_layer(params, name, _PERM if name == 'gc1' else None)
    if name == 'gc1':
      uv = _uv_call(h, None, None, wuv, b3, embed=True)
    else:
      g = params[prev_norm + '_g']
      b = params[prev_norm + '_b']
      mu = stats[0] / N
      var = stats[1] / N - mu * mu
      inv = lax.rsqrt(var + 1e-5)
      scale = (g * inv).reshape(1, -1)
      shift = (b - mu * g * inv).reshape(1, -1)
      uv = _uv_call(h, scale, shift, wuv, b3, embed=False)
    nchunk = dh // CSC
    if nchunk not in edge_kernels:
      edge_kernels[nchunk] = _make_edge_kernel(NPAD, nchunk, CSC, NC, NS, K, NB)
    sp = edge_kernels[nchunk](uv, srcg, dstg)
    h, stats = _out_call(sp, cnt, params[name + '_W2'],
                         params[name + '_b2'].reshape(1, -1))
    prev_norm = norm
  return h[:N]


# speedup vs baseline: 2.2442x; 1.3783x over previous
import functools

import numpy as np
import jax
import jax.numpy as jnp
from jax import lax
from jax.experimental import pallas as pl
from jax.experimental.pallas import tpu as pltpu
from jax.experimental.pallas import tpu_sc as plsc

N = 10000
NPAD = 10240
E = 160000
NC = 2
NS = 16
NW = NC * NS
C = 128
CSC = 64
K = 125
EW = E // NW
NB = EW // K
MB = 1280
NMB = NPAD // MB


def _make_edge_kernel(n, nchunk, c, nc, ns, k, nb):
  npr = n // ns
  zr = 128
  nz = npr // zr
  assert npr % zr == 0
  mesh = plsc.VectorSubcoreMesh(
      core_axis_name="c", subcore_axis_name="s",
      num_cores=nc, num_subcores=ns)

  @functools.partial(
      pl.kernel,
      out_type=jax.ShapeDtypeStruct((nc, nchunk, n, c), jnp.float32),
      mesh=mesh,
      compiler_params=pltpu.CompilerParams(use_tc_tiling_on_sc=False),
      scratch_types=[
          pltpu.VMEM((nb, 1, k), jnp.int32),
          pltpu.VMEM((nb, 1, k), jnp.int32),
          pltpu.VMEM((k, c), jnp.float32),
          pltpu.VMEM((k, c), jnp.float32),
          pltpu.VMEM((k, c), jnp.float32),
          pltpu.VMEM((k, c), jnp.float32),
          pltpu.VMEM((k, c), jnp.float32),
          pltpu.VMEM((zr, c), jnp.float32),
          pltpu.VMEM_SHARED((n, c), jnp.float32),
          pltpu.SemaphoreType.DMA,
          pltpu.SemaphoreType.DMA,
          pltpu.SemaphoreType.DMA,
          pltpu.SemaphoreType.DMA,
      ],
  )
  def edge_kernel(uv, srcg, dstg, out, srcv, dstv, ub0, vb0, ub1, vb1, rb,
                  zbuf, s_sh, semu0, semv0, semu1, semv1):
    cid = lax.axis_index("c")
    sid = lax.axis_index("s")
    wid = sid * nc + cid
    pltpu.sync_copy(srcg.at[wid], srcv)
    pltpu.sync_copy(dstg.at[wid], dstv)
    bufs = ((ub0, vb0, semu0, semv0), (ub1, vb1, semu1, semv1))

    def zrow(i, carry):
      for c8 in range(c // 16):
        zbuf[i, pl.ds(c8 * 16, 16)] = jnp.zeros((16,), jnp.float32)
      return carry
    lax.fori_loop(0, zr, zrow, 0)

    def fire(ch, j, b):
      ub, vb, semu, semv = bufs[b]
      pltpu.async_copy(uv.at[ch].at[dstv.at[j, 0]], ub, semu)
      pltpu.async_copy(uv.at[nchunk + ch].at[srcv.at[j, 0]], vb, semv)

    def drain(ch, j, b):
      ub, vb, semu, semv = bufs[b]
      pltpu.make_async_copy(uv.at[ch].at[dstv.at[j, 0]], ub, semu).wait()
      pltpu.make_async_copy(uv.at[ch].at[srcv.at[j, 0]], vb, semv).wait()

    def chunk(ch, ccarry):
      for z in range(nz):
        pltpu.sync_copy(zbuf, s_sh.at[pl.ds(sid * npr + z * zr, zr)])
      plsc.subcore_barrier()

      fire(ch, 0, 0)

      @pl.loop(0, nb, step=2)
      def batch2(j):
        for b in range(2):
          jj = j + b
          ub, vb, _, _ = bufs[b]
          nxt = jnp.where(jj + 1 == nb, 0, jj + 1)
          fire(ch, nxt, (b + 1) % 2)
          drain(ch, jj, b)

          def rows(i, rcarry):
            for r5 in range(5):
              ii = i * 5 + r5
              for c8 in range(c // 16):
                s = pl.ds(c8 * 16, 16)
                rb[ii, s] = jnp.maximum(ub[ii, s] + vb[ii, s], 0.0)
            return rcarry
          lax.fori_loop(0, k // 5, rows, 0)
          pltpu.sync_copy(rb, s_sh.at[dstv.at[jj, 0]], add=True)

      drain(ch, 0, 0)
      plsc.subcore_barrier()

      pltpu.sync_copy(s_sh.at[pl.ds(sid * npr, npr)],
                      out.at[cid, ch, pl.ds(sid * npr, npr)])
      plsc.subcore_barrier()
      return ccarry
    lax.fori_loop(0, nchunk, chunk, 0)

  return edge_kernel


def _uv_embed_body(x_ref, w_ref, b_ref, o_ref):
  xb = x_ref[...]
  p9 = xb[:, 0:9]
  feats = [xb]
  feats += [jnp.sin(p9 * (2.0 ** f)) for f in range(10)]
  feats += [jnp.cos(p9 * (2.0 ** f)) for f in range(10)]
  h = jnp.concatenate(feats, axis=1)
  r = jnp.dot(h, w_ref[...], preferred_element_type=jnp.float32, precision=lax.Precision.HIGHEST) + b_ref[0]
  o_ref[0] = r[:, :CSC]
  o_ref[1] = r[:, CSC:]


def _uv_norm_body(h_ref, sc_ref, sh_ref, w_ref, b_ref, o_ref):
  h = jnp.maximum(h_ref[...] * sc_ref[...] + sh_ref[...], 0.0)
  r = jnp.dot(h, w_ref[...], preferred_element_type=jnp.float32, precision=lax.Precision.HIGHEST) + b_ref[0]
  o_ref[0] = r[:, :CSC]
  o_ref[1] = r[:, CSC:]


def _uv_call(h, scale, shift, w, b3, embed):
  n, d = h.shape
  nchunk2 = w.shape[1] // C
  if embed:
    body = _uv_embed_body
    in_arrays = (h, w, b3)
    in_specs = [
        pl.BlockSpec((MB, d), lambda m, j: (m, 0)),
        pl.BlockSpec((w.shape[0], C), lambda m, j: (0, j)),
        pl.BlockSpec((1, 1, C), lambda m, j: (j, 0, 0)),
    ]
  else:
    body = _uv_norm_body
    in_arrays = (h, scale, shift, w, b3)
    in_specs = [
        pl.BlockSpec((MB, d), lambda m, j: (m, 0)),
        pl.BlockSpec((1, d), lambda m, j: (0, 0)),
        pl.BlockSpec((1, d), lambda m, j: (0, 0)),
        pl.BlockSpec((d, C), lambda m, j: (0, j)),
        pl.BlockSpec((1, 1, C), lambda m, j: (j, 0, 0)),
    ]
  return pl.pallas_call(
      body,
      grid=(NMB, nchunk2),
      in_specs=in_specs,
      out_specs=pl.BlockSpec((2, MB, CSC), lambda m, j: (j, m, 0)),
      out_shape=jax.ShapeDtypeStruct((2 * nchunk2, n, CSC), jnp.float32),
  )(*in_arrays)


def _make_out_body(nchunk, dout):
  def body(s_ref, cnt_ref, w_ref, b_ref, o_ref, st_ref):
    acc = jnp.zeros((MB, dout), jnp.float32)
    for ch in range(nchunk // 2):
      t = jnp.concatenate(
          [s_ref[0, 2 * ch] + s_ref[1, 2 * ch],
           s_ref[0, 2 * ch + 1] + s_ref[1, 2 * ch + 1]], axis=1)
      acc = acc + jnp.dot(t, w_ref[pl.ds(ch * C, C), :],
                          preferred_element_type=jnp.float32,
                          precision=lax.Precision.HIGHEST)
    cnt = cnt_ref[...]
    outb = (acc + cnt * b_ref[...]) / jnp.maximum(cnt, 1.0)
    o_ref[...] = outb
    s0 = jnp.sum(outb, axis=0, keepdims=True)
    s1 = jnp.sum(outb * outb, axis=0, keepdims=True)
    st = jnp.concatenate([s0, s1], axis=0)

    @pl.when(pl.program_id(0) == 0)
    def _init():
      st_ref[...] = st

    @pl.when(pl.program_id(0) != 0)
    def _acc():
      st_ref[...] = st_ref[...] + st
  return body


def _out_call(sp, cnt, w2, b2):
  nc, nchunk, n, c = sp.shape
  dh, dout = w2.shape
  return pl.pallas_call(
      _make_out_body(nchunk, dout),
      grid=(NMB,),
      in_specs=[
          pl.BlockSpec((nc, nchunk, MB, c), lambda m: (0, 0, m, 0)),
          pl.BlockSpec((MB, 1), lambda m: (m, 0)),
          pl.BlockSpec((dh, dout), lambda m: (0, 0)),
          pl.BlockSpec((1, dout), lambda m: (0, 0)),
      ],
      out_specs=[
          pl.BlockSpec((MB, dout), lambda m: (m, 0)),
          pl.BlockSpec((2, dout), lambda m: (0, 0)),
      ],
      out_shape=[
          jax.ShapeDtypeStruct((n, dout), jnp.float32),
          jax.ShapeDtypeStruct((2, dout), jnp.float32),
      ],
  )(sp, cnt, w2, b2)


def _build_perm():
  perm = np.zeros(196, np.int32)
  for m in range(196):
    if m < 9:
      b, r = divmod(m, 3)
      perm[m] = b * 63 + r
    elif m < 16:
      perm[m] = 189 + (m - 9)
    elif m < 106:
      f, kk = divmod(m - 16, 9)
      b, r = divmod(kk, 3)
      perm[m] = b * 63 + 3 + f * 3 + r
    else:
      f, kk = divmod(m - 106, 9)
      b, r = divmod(kk, 3)
      perm[m] = b * 63 + 33 + f * 3 + r
  return perm

_PERM = _build_perm()


def _prep_layer(params, name, perm=None):
  w1 = params[name + '_W1']
  d = w1.shape[0] // 2
  dh = w1.shape[1]
  wa, wb = w1[:d], w1[d:]
  wuv = jnp.concatenate([wa - wb, wb], axis=1)
  if perm is not None:
    wuv = wuv[perm, :]
  b1 = params[name + '_b1']
  bcat = jnp.concatenate([b1, jnp.zeros_like(b1)])
  b3 = bcat.reshape(2 * dh // C, 1, C)
  return wuv, b3, dh


def kernel(x, edge_index, batch, params):
  del batch
  src = edge_index[0].astype(jnp.int32)
  dst = edge_index[1].astype(jnp.int32)
  srcg = src.reshape(NW, NB, 1, K)
  dstg = dst.reshape(NW, NB, 1, K)

  uv_cnt = jnp.concatenate(
      [jnp.zeros((1, NPAD, CSC), jnp.float32),
       jnp.ones((1, NPAD, CSC), jnp.float32)],
      axis=0)
  cnt_kernel = _make_edge_kernel(NPAD, 1, CSC, NC, NS, K, NB)
  cp = cnt_kernel(uv_cnt, srcg, dstg)
  cnt = cp[0, 0, :, 0:1] + cp[1, 0, :, 0:1]

  layer_specs = [
      ('gc1', 'n1'), ('gc2', 'n2'), ('gc3', 'n3'), ('gc4', 'n4'),
      ('gc5', None),
  ]
  edge_kernels = {}
  h = jnp.pad(x, ((0, NPAD - N), (0, 0)))
  stats = None
  prev_norm = None
  for name, norm in layer_specs:
    wuv, b3, dh = _prep_layer(params, name, _PERM if name == 'gc1' else None)
    if name == 'gc1':
      uv = _uv_call(h, None, None, wuv, b3, embed=True)
    else:
      g = params[prev_norm + '_g']
      b = params[prev_norm + '_b']
      mu = stats[0] / N
      var = stats[1] / N - mu * mu
      inv = lax.rsqrt(var + 1e-5)
      scale = (g * inv).reshape(1, -1)
      shift = (b - mu * g * inv).reshape(1, -1)
      uv = _uv_call(h, scale, shift, wuv, b3, embed=False)
    nchunk = dh // CSC
    if nchunk not in edge_kernels:
      edge_kernels[nchunk] = _make_edge_kernel(NPAD, nchunk, CSC, NC, NS, K, NB)
    sp = edge_kernels[nchunk](uv, srcg, dstg)
    h, stats = _out_call(sp, cnt, params[name + '_W2'],
                         params[name + '_b2'].reshape(1, -1))
    prev_norm = norm
  return h[:N]


# speedup vs baseline: 2.4794x; 1.1048x over previous
import functools

import numpy as np
import jax
import jax.numpy as jnp
from jax import lax
from jax.experimental import pallas as pl
from jax.experimental.pallas import tpu as pltpu
from jax.experimental.pallas import tpu_sc as plsc

N = 10000
NPAD = 10240
E = 160000
NC = 2
NS = 16
NW = NC * NS
C = 128
CSC = 64
K = 125
EW = E // NW
NB = EW // K
MB = 1280
NMB = NPAD // MB


def _make_edge_kernel(n, nchunk, c, nc, ns, k, nb):
  npr = n // ns
  zr = 128
  nz = npr // zr
  assert npr % zr == 0
  mesh = plsc.VectorSubcoreMesh(
      core_axis_name="c", subcore_axis_name="s",
      num_cores=nc, num_subcores=ns)

  @functools.partial(
      pl.kernel,
      out_type=jax.ShapeDtypeStruct((nc, nchunk, n, c), jnp.float32),
      mesh=mesh,
      compiler_params=pltpu.CompilerParams(use_tc_tiling_on_sc=False),
      scratch_types=[
          pltpu.VMEM((nb, 1, k), jnp.int32),
          pltpu.VMEM((nb, 1, k), jnp.int32),
          pltpu.VMEM((k, c), jnp.float32),
          pltpu.VMEM((k, c), jnp.float32),
          pltpu.VMEM((k, c), jnp.float32),
          pltpu.VMEM((k, c), jnp.float32),
          pltpu.VMEM((k, c), jnp.float32),
          pltpu.VMEM((zr, c), jnp.float32),
          pltpu.VMEM_SHARED((n, c), jnp.float32),
          pltpu.SemaphoreType.DMA,
          pltpu.SemaphoreType.DMA,
          pltpu.SemaphoreType.DMA,
          pltpu.SemaphoreType.DMA,
      ],
  )
  def edge_kernel(uv, srcg, dstg, out, srcv, dstv, ub0, vb0, ub1, vb1, rb,
                  zbuf, s_sh, semu0, semv0, semu1, semv1):
    cid = lax.axis_index("c")
    sid = lax.axis_index("s")
    wid = sid * nc + cid
    pltpu.sync_copy(srcg.at[wid], srcv)
    pltpu.sync_copy(dstg.at[wid], dstv)
    bufs = ((ub0, vb0, semu0, semv0), (ub1, vb1, semu1, semv1))

    def zrow(i, carry):
      for c8 in range(c // 16):
        zbuf[i, pl.ds(c8 * 16, 16)] = jnp.zeros((16,), jnp.float32)
      return carry
    lax.fori_loop(0, zr, zrow, 0)

    def fire(ch, j, b):
      ub, vb, semu, semv = bufs[b]
      pltpu.async_copy(uv.at[ch].at[dstv.at[j, 0]], ub, semu)
      pltpu.async_copy(uv.at[nchunk + ch].at[srcv.at[j, 0]], vb, semv)

    def drain(ch, j, b):
      ub, vb, semu, semv = bufs[b]
      pltpu.make_async_copy(uv.at[ch].at[dstv.at[j, 0]], ub, semu).wait()
      pltpu.make_async_copy(uv.at[ch].at[srcv.at[j, 0]], vb, semv).wait()

    def chunk(ch, ccarry):
      for z in range(nz):
        pltpu.sync_copy(zbuf, s_sh.at[pl.ds(sid * npr + z * zr, zr)])
      plsc.subcore_barrier()

      fire(ch, 0, 0)

      @pl.loop(0, nb, step=2)
      def batch2(j):
        for b in range(2):
          jj = j + b
          ub, vb, _, _ = bufs[b]
          nxt = jnp.where(jj + 1 == nb, 0, jj + 1)
          fire(ch, nxt, (b + 1) % 2)
          drain(ch, jj, b)

          def rows(i, rcarry):
            for r5 in range(5):
              ii = i * 5 + r5
              for c8 in range(c // 16):
                s = pl.ds(c8 * 16, 16)
                rb[ii, s] = jnp.maximum(ub[ii, s] + vb[ii, s], 0.0)
            return rcarry
          lax.fori_loop(0, k // 5, rows, 0)
          pltpu.sync_copy(rb, s_sh.at[dstv.at[jj, 0]], add=True)

      drain(ch, 0, 0)
      plsc.subcore_barrier()

      pltpu.sync_copy(s_sh.at[pl.ds(sid * npr, npr)],
                      out.at[cid, ch, pl.ds(sid * npr, npr)])
      plsc.subcore_barrier()
      return ccarry
    lax.fori_loop(0, nchunk, chunk, 0)

  return edge_kernel


def _uv_embed_body(x_ref, w_ref, b_ref, o_ref):
  xb = x_ref[...]
  p9 = xb[:, 0:9]
  xs = jnp.concatenate([p9 * (2.0 ** f) for f in range(10)], axis=1)
  h = jnp.concatenate([xb, jnp.sin(xs), jnp.cos(xs)], axis=1)
  r = jnp.dot(h, w_ref[...], preferred_element_type=jnp.float32,
              precision=lax.Precision.HIGHEST) + b_ref[0]
  nchunk2 = r.shape[1] // C
  for j in range(2 * nchunk2):
    o_ref[j] = r[:, j * CSC:(j + 1) * CSC]


def _uv_norm_body(h_ref, sc_ref, sh_ref, w_ref, b_ref, o_ref):
  h = jnp.maximum(h_ref[...] * sc_ref[...] + sh_ref[...], 0.0)
  r = jnp.dot(h, w_ref[...], preferred_element_type=jnp.float32, precision=lax.Precision.HIGHEST) + b_ref[0]
  o_ref[0] = r[:, :CSC]
  o_ref[1] = r[:, CSC:]


def _uv_call(h, scale, shift, w, b3, embed):
  n, d = h.shape
  nchunk2 = w.shape[1] // C
  if embed:
    b_flat = b3.reshape(1, 1, -1)
    return pl.pallas_call(
        _uv_embed_body,
        grid=(NMB,),
        in_specs=[
            pl.BlockSpec((MB, d), lambda m: (m, 0)),
            pl.BlockSpec(w.shape, lambda m: (0, 0)),
            pl.BlockSpec(b_flat.shape, lambda m: (0, 0, 0)),
        ],
        out_specs=pl.BlockSpec((2 * nchunk2, MB, CSC), lambda m: (0, m, 0)),
        out_shape=jax.ShapeDtypeStruct((2 * nchunk2, n, CSC), jnp.float32),
    )(h, w, b_flat)
  else:
    body = _uv_norm_body
    in_arrays = (h, scale, shift, w, b3)
    in_specs = [
        pl.BlockSpec((MB, d), lambda m, j: (m, 0)),
        pl.BlockSpec((1, d), lambda m, j: (0, 0)),
        pl.BlockSpec((1, d), lambda m, j: (0, 0)),
        pl.BlockSpec((d, C), lambda m, j: (0, j)),
        pl.BlockSpec((1, 1, C), lambda m, j: (j, 0, 0)),
    ]
  return pl.pallas_call(
      body,
      grid=(NMB, nchunk2),
      in_specs=in_specs,
      out_specs=pl.BlockSpec((2, MB, CSC), lambda m, j: (j, m, 0)),
      out_shape=jax.ShapeDtypeStruct((2 * nchunk2, n, CSC), jnp.float32),
  )(*in_arrays)


def _make_out_body(nchunk, dout):
  def body(s_ref, cnt_ref, w_ref, b_ref, o_ref, st_ref):
    acc = jnp.zeros((MB, dout), jnp.float32)
    for ch in range(nchunk // 2):
      t = jnp.concatenate(
          [s_ref[0, 2 * ch] + s_ref[1, 2 * ch],
           s_ref[0, 2 * ch + 1] + s_ref[1, 2 * ch + 1]], axis=1)
      acc = acc + jnp.dot(t, w_ref[pl.ds(ch * C, C), :],
                          preferred_element_type=jnp.float32,
                          precision=lax.Precision.HIGHEST)
    cnt = cnt_ref[...]
    outb = (acc + cnt * b_ref[...]) / jnp.maximum(cnt, 1.0)
    o_ref[...] = outb
    s0 = jnp.sum(outb, axis=0, keepdims=True)
    s1 = jnp.sum(outb * outb, axis=0, keepdims=True)
    st = jnp.concatenate([s0, s1], axis=0)

    @pl.when(pl.program_id(0) == 0)
    def _init():
      st_ref[...] = st

    @pl.when(pl.program_id(0) != 0)
    def _acc():
      st_ref[...] = st_ref[...] + st
  return body


def _out_call(sp, cnt, w2, b2):
  nc, nchunk, n, c = sp.shape
  dh, dout = w2.shape
  return pl.pallas_call(
      _make_out_body(nchunk, dout),
      grid=(NMB,),
      in_specs=[
          pl.BlockSpec((nc, nchunk, MB, c), lambda m: (0, 0, m, 0)),
          pl.BlockSpec((MB, 1), lambda m: (m, 0)),
          pl.BlockSpec((dh, dout), lambda m: (0, 0)),
          pl.BlockSpec((1, dout), lambda m: (0, 0)),
      ],
      out_specs=[
          pl.BlockSpec((MB, dout), lambda m: (m, 0)),
          pl.BlockSpec((2, dout), lambda m: (0, 0)),
      ],
      out_shape=[
          jax.ShapeDtypeStruct((n, dout), jnp.float32),
          jax.ShapeDtypeStruct((2, dout), jnp.float32),
      ],
  )(sp, cnt, w2, b2)


def _build_perm():
  perm = np.zeros(196, np.int32)
  for m in range(196):
    if m < 9:
      b, r = divmod(m, 3)
      perm[m] = b * 63 + r
    elif m < 16:
      perm[m] = 189 + (m - 9)
    elif m < 106:
      f, kk = divmod(m - 16, 9)
      b, r = divmod(kk, 3)
      perm[m] = b * 63 + 3 + f * 3 + r
    else:
      f, kk = divmod(m - 106, 9)
      b, r = divmod(kk, 3)
      perm[m] = b * 63 + 33 + f * 3 + r
  return perm

_PERM = _build_perm()


def _prep_layer(params, name, perm=None):
  w1 = params[name + '_W1']
  d = w1.shape[0] // 2
  dh = w1.shape[1]
  wa, wb = w1[:d], w1[d:]
  wuv = jnp.concatenate([wa - wb, wb], axis=1)
  if perm is not None:
    wuv = wuv[perm, :]
  b1 = params[name + '_b1']
  bcat = jnp.concatenate([b1, jnp.zeros_like(b1)])
  b3 = bcat.reshape(2 * dh // C, 1, C)
  return wuv, b3, dh


def kernel(x, edge_index, batch, params):
  del batch
  src = edge_index[0].astype(jnp.int32)
  dst = edge_index[1].astype(jnp.int32)
  srcg = src.reshape(NW, NB, 1, K)
  dstg = dst.reshape(NW, NB, 1, K)

  uv_cnt = jnp.concatenate(
      [jnp.zeros((1, NPAD, CSC), jnp.float32),
       jnp.ones((1, NPAD, CSC), jnp.float32)],
      axis=0)
  cnt_kernel = _make_edge_kernel(NPAD, 1, CSC, NC, NS, K, NB)
  cp = cnt_kernel(uv_cnt, srcg, dstg)
  cnt = cp[0, 0, :, 0:1] + cp[1, 0, :, 0:1]

  layer_specs = [
      ('gc1', 'n1'), ('gc2', 'n2'), ('gc3', 'n3'), ('gc4', 'n4'),
      ('gc5', None),
  ]
  edge_kernels = {}
  h = jnp.pad(x, ((0, NPAD - N), (0, 0)))
  stats = None
  prev_norm = None
  for name, norm in layer_specs:
    wuv, b3, dh = _prep_layer(params, name, _PERM if name == 'gc1' else None)
    if name == 'gc1':
      uv = _uv_call(h, None, None, wuv, b3, embed=True)
    else:
      g = params[prev_norm + '_g']
      b = params[prev_norm + '_b']
      mu = stats[0] / N
      var = stats[1] / N - mu * mu
      inv = lax.rsqrt(var + 1e-5)
      scale = (g * inv).reshape(1, -1)
      shift = (b - mu * g * inv).reshape(1, -1)
      uv = _uv_call(h, scale, shift, wuv, b3, embed=False)
    nchunk = dh // CSC
    if nchunk not in edge_kernels:
      edge_kernels[nchunk] = _make_edge_kernel(NPAD, nchunk, CSC, NC, NS, K, NB)
    sp = edge_kernels[nchunk](uv, srcg, dstg)
    h, stats = _out_call(sp, cnt, params[name + '_W2'],
                         params[name + '_b2'].reshape(1, -1))
    prev_norm = norm
  return h[:N]
